# baseline (speedup 1.0000x reference)
import jax
import jax.numpy as jnp
from jax.experimental import pallas as pl
from jax.experimental.pallas import tpu as pltpu

_D = 768
_K = 1024
_TILE_T = 2048


def _vq_block(x_ref, w_ref, b_ref, cb_ref, out_ref, cbn_ref):
    cb = cb_ref[...]

    @pl.when(pl.program_id(0) == 0)
    def _():
        cbn_ref[...] = jnp.sum(cb * cb, axis=1, keepdims=True)

    x = x_ref[0]
    w = w_ref[...]
    y = jnp.dot(w, x, preferred_element_type=jnp.float32) + b_ref[...]
    s = jnp.dot(cb, y, preferred_element_type=jnp.float32)
    xn = jnp.sum(y * y, axis=0, keepdims=True)
    dist = (xn - 2.0 * s) + cbn_ref[...]
    mval = jnp.min(dist, axis=0, keepdims=True)
    iota = jax.lax.broadcasted_iota(jnp.int32, dist.shape, 0)
    cand = jnp.where(dist == mval, iota, _K)
    out_ref[...] = jnp.min(cand, axis=0)[None, :]


def kernel(ssl_content, proj_w, proj_b, codebook):
    t_len = ssl_content.shape[2]
    b2 = proj_b[:, None]
    return pl.pallas_call(
        _vq_block,
        grid=(t_len // _TILE_T,),
        in_specs=[
            pl.BlockSpec((1, _D, _TILE_T), lambda i: (0, 0, i)),
            pl.BlockSpec((_D, _D), lambda i: (0, 0)),
            pl.BlockSpec((_D, 1), lambda i: (0, 0)),
            pl.BlockSpec((_K, _D), lambda i: (0, 0)),
        ],
        out_specs=pl.BlockSpec((1, _TILE_T), lambda i: (0, i)),
        out_shape=jax.ShapeDtypeStruct((1, t_len), jnp.int32),
        scratch_shapes=[pltpu.VMEM((_K, 1), jnp.float32)],
    )(ssl_content, proj_w, b2, codebook)


# speedup vs baseline: 1.0611x; 1.0611x over previous
import jax
import jax.numpy as jnp
from jax.experimental import pallas as pl
from jax.experimental.pallas import tpu as pltpu

_D = 768
_K = 1024
_TILE_T = 2048
_HALF = _TILE_T // 2


def _vq_block(x_ref, w_ref, b_ref, cb_ref, out_ref, cbn_ref):
    cb = cb_ref[...]

    @pl.when(pl.program_id(0) == 0)
    def _():
        cbn_ref[...] = jnp.sum(cb * cb, axis=1, keepdims=True)

    w = w_ref[...]
    for h in range(_TILE_T // _HALF):
        x = x_ref[0, :, h * _HALF:(h + 1) * _HALF]
        y = jnp.dot(w, x, preferred_element_type=jnp.float32) + b_ref[...]
        s = jnp.dot(cb, y, preferred_element_type=jnp.float32)
        xn = jnp.sum(y * y, axis=0, keepdims=True)
        dist = (xn - 2.0 * s) + cbn_ref[...]
        idx = jnp.argmin(dist, axis=0)[None, :].astype(jnp.int32)
        out_ref[:, h * _HALF:(h + 1) * _HALF] = idx


def kernel(ssl_content, proj_w, proj_b, codebook):
    t_len = ssl_content.shape[2]
    b2 = proj_b[:, None]
    return pl.pallas_call(
        _vq_block,
        grid=(t_len // _TILE_T,),
        in_specs=[
            pl.BlockSpec((1, _D, _TILE_T), lambda i: (0, 0, i)),
            pl.BlockSpec((_D, _D), lambda i: (0, 0)),
            pl.BlockSpec((_D, 1), lambda i: (0, 0)),
            pl.BlockSpec((_K, _D), lambda i: (0, 0)),
        ],
        out_specs=pl.BlockSpec((1, _TILE_T), lambda i: (0, i)),
        out_shape=jax.ShapeDtypeStruct((1, t_len), jnp.int32),
        scratch_shapes=[pltpu.VMEM((_K, 1), jnp.float32)],
    )(ssl_content, proj_w, b2, codebook)
